# baseline (speedup 1.0000x reference)
import functools

import jax
import jax.numpy as jnp
from jax import lax
from jax.experimental import pallas as pl
from jax.experimental.pallas import tpu as pltpu
from jax.experimental.pallas import tpu_sc as plsc

N = 100000
F = 128
NUM_SEG = 16
RATIO = 4

NC = 2
NS = 16
NW = NC * NS
L = 16

R = 400
N_SC = 51200
NB = N_SC // R
BLOCKS_PER_W = NB // NW

TC_CHUNK = 800
N_TC = N - N_SC
TC_STEPS = N_TC // TC_CHUNK


def _sc_segment_sum(x, batch):
    mesh = plsc.VectorSubcoreMesh(core_axis_name="c", subcore_axis_name="s")

    @functools.partial(
        pl.kernel,
        mesh=mesh,
        out_type=jax.ShapeDtypeStruct((NW, NUM_SEG, F), jnp.float32),
        compiler_params=pltpu.CompilerParams(needs_layout_passes=False),
        scratch_types=[
            pltpu.VMEM((R, F), jnp.float32),
            pltpu.VMEM((R, F), jnp.float32),
            pltpu.VMEM((R,), jnp.int32),
            pltpu.VMEM((R,), jnp.int32),
            pltpu.VMEM((NUM_SEG, F), jnp.float32),
            pltpu.SemaphoreType.DMA,
            pltpu.SemaphoreType.DMA,
        ],
    )
    def body(x_hbm, b_hbm, out_hbm, xb0, xb1, bb0, bb1, acc, sem0, sem1):
        wid = lax.axis_index("s") * NC + lax.axis_index("c")
        iota = lax.broadcasted_iota(jnp.int32, (L,), 0)
        zeros = jnp.zeros((L,), jnp.float32)
        xbs, bbs, sems = (xb0, xb1), (bb0, bb1), (sem0, sem1)

        def zero_body(i, _):
            for c in range(F // L):
                acc[i, pl.ds(c * L, L)] = zeros
            return 0

        lax.fori_loop(0, NUM_SEG, zero_body, 0)

        def dma_start(b, s):
            pltpu.async_copy(x_hbm.at[pl.ds(b * R, R), :], xbs[s], sems[s])
            pltpu.async_copy(b_hbm.at[pl.ds(b * R, R)], bbs[s], sems[s])

        def dma_wait(b, s):
            pltpu.make_async_copy(
                x_hbm.at[pl.ds(b * R, R), :], xbs[s], sems[s]).wait()
            pltpu.make_async_copy(
                b_hbm.at[pl.ds(b * R, R)], bbs[s], sems[s]).wait()

        def _tree_group_sum(xb, row0, c):
            v = [xb[row0 + j, pl.ds(c * L, L)] for j in range(L)]
            while len(v) > 1:
                v = [v[i] + v[i + 1] for i in range(0, len(v), 2)]
            return v[0]

        def compute(s):
            xb, bb = xbs[s], bbs[s]
            bhead = bb[pl.ds(0, L)]
            btail = bb[pl.ds(R - L, L)]
            block_uniform = jnp.all(bhead == btail)

            @pl.when(block_uniform)
            def _uniform_block():
                def gb(g, carry):
                    return tuple(
                        carry[c] + _tree_group_sum(xb, g * L, c)
                        for c in range(F // L))

                tot = lax.fori_loop(
                    0, R // L, gb,
                    tuple(jnp.zeros((L,), jnp.float32) for _ in range(F // L)))
                for c in range(F // L):
                    plsc.addupdate_scatter(acc, [bhead, iota + c * L], tot[c])

            @pl.when(jnp.logical_not(block_uniform))
            def _mixed_block():
                def group_body(g, _):
                    row0 = g * L
                    bvec = bb[pl.ds(row0, L)]
                    b0 = bvec.at[jnp.zeros((L,), jnp.int32)].get(
                        mode="promise_in_bounds")
                    uniform = jnp.all(bvec == b0)

                    @pl.when(uniform)
                    def _fast():
                        for c in range(F // L):
                            v = _tree_group_sum(xb, row0, c)
                            plsc.addupdate_scatter(acc, [bvec, iota + c * L], v)

                    @pl.when(jnp.logical_not(uniform))
                    def _slow():
                        for j in range(L):
                            bj = bvec.at[jnp.full((L,), j, jnp.int32)].get(
                                mode="promise_in_bounds")
                            for c in range(F // L):
                                v = xb[row0 + j, pl.ds(c * L, L)]
                                plsc.addupdate_scatter(
                                    acc, [bj, iota + c * L], v)

                    return 0

                lax.fori_loop(0, R // L, group_body, 0)

        dma_start(wid, 0)
        for k in range(BLOCKS_PER_W):
            if (k + 1) < BLOCKS_PER_W:
                dma_start(wid + NW * (k + 1), (k + 1) % 2)
            dma_wait(wid + NW * k, k % 2)
            compute(k % 2)

        pltpu.sync_copy(acc, out_hbm.at[wid])

    return body(x, batch)


def _tc_partial(x, batch3):
    def body(b_ref, x_ref, out_ref):
        i = pl.program_id(0)

        @pl.when(i == 0)
        def _init():
            out_ref[...] = jnp.zeros_like(out_ref)

        seg = lax.broadcasted_iota(jnp.int32, (NUM_SEG, TC_CHUNK), 0)
        oh = (b_ref[0, 0, :][None, :] == seg).astype(jnp.float32)
        out_ref[...] += lax.dot_general(
            oh, x_ref[...], (((1,), (0,)), ((), ())),
            preferred_element_type=jnp.float32)

    return pl.pallas_call(
        body,
        grid=(TC_STEPS,),
        in_specs=[
            pl.BlockSpec((1, 1, TC_CHUNK),
                         lambda i: (N_SC // TC_CHUNK + i, 0, 0)),
            pl.BlockSpec((TC_CHUNK, F), lambda i: (N_SC // TC_CHUNK + i, 0)),
        ],
        out_specs=pl.BlockSpec((NUM_SEG, F), lambda i: (0, 0)),
        out_shape=jax.ShapeDtypeStruct((NUM_SEG, F), jnp.float32),
    )(batch3, x)


def _tc_combine(sc_partials, tc_partial, seed_nodes, W_rel, W_root, b_rel2):
    def body(p_ref, t_ref, seed_ref, wrel_ref, wroot_ref, brel_ref, out_ref):
        S = jnp.sum(p_ref[...], axis=0) + t_ref[...]
        A = lax.dot_general(S, wrel_ref[...], (((1,), (1,)), ((), ())),
                            preferred_element_type=jnp.float32)
        B = lax.dot_general(seed_ref[...], wroot_ref[...], (((1,), (1,)), ((), ())),
                            preferred_element_type=jnp.float32)
        out_ref[...] = (A[:, None, :] + B[None, :, :]
                        + brel_ref[...][None, :, :])

    return pl.pallas_call(
        body,
        out_shape=jax.ShapeDtypeStruct((NUM_SEG, RATIO, F), jnp.float32),
    )(sc_partials, tc_partial, seed_nodes, W_rel, W_root, b_rel2)


def kernel(x, batch, seed_nodes, W_rel, W_root, b_rel):
    batch = batch.astype(jnp.int32)
    sc_partials = _sc_segment_sum(x, batch)
    tc_part = _tc_partial(x, batch.reshape(N // TC_CHUNK, 1, TC_CHUNK))
    out3 = _tc_combine(sc_partials, tc_part, seed_nodes, W_rel, W_root,
                       b_rel.reshape(1, F))
    out = out3.reshape(NUM_SEG * RATIO, F)
    new_batchidx = jnp.repeat(jnp.arange(NUM_SEG, dtype=jnp.int32), RATIO)
    return out, new_batchidx


# speedup vs baseline: 1.1632x; 1.1632x over previous
import functools

import jax
import jax.numpy as jnp
import numpy as np
from jax import lax
from jax.experimental import pallas as pl
from jax.experimental.pallas import tpu as pltpu
from jax.experimental.pallas import tpu_sc as plsc

N = 100000
F = 128
NUM_SEG = 16
RATIO = 4

NC = 2
NS = 16
NW = NC * NS
L = 16

R = 400
N_SC = 64000
NB = N_SC // R
BLOCKS_PER_W = NB // NW

TC_CHUNK = 2000
N_TC = N - N_SC
TC_STEPS = N_TC // TC_CHUNK


def _sc_segment_sum(x, batch):
    mesh = plsc.VectorSubcoreMesh(core_axis_name="c", subcore_axis_name="s")

    @functools.partial(
        pl.kernel,
        mesh=mesh,
        out_type=jax.ShapeDtypeStruct((NW, NUM_SEG, F), jnp.float32),
        compiler_params=pltpu.CompilerParams(needs_layout_passes=False),
        scratch_types=[
            pltpu.VMEM((R, F), jnp.float32),
            pltpu.VMEM((R, F), jnp.float32),
            pltpu.VMEM((R,), jnp.int32),
            pltpu.VMEM((R,), jnp.int32),
            pltpu.VMEM((NUM_SEG, F), jnp.float32),
            pltpu.SemaphoreType.DMA,
            pltpu.SemaphoreType.DMA,
        ],
    )
    def body(x_hbm, b_hbm, out_hbm, xb0, xb1, bb0, bb1, acc, sem0, sem1):
        wid = lax.axis_index("s") * NC + lax.axis_index("c")
        iota = lax.broadcasted_iota(jnp.int32, (L,), 0)
        zeros = jnp.zeros((L,), jnp.float32)
        xbs, bbs, sems = (xb0, xb1), (bb0, bb1), (sem0, sem1)

        def zero_body(i, _):
            for c in range(F // L):
                acc[i, pl.ds(c * L, L)] = zeros
            return 0

        lax.fori_loop(0, NUM_SEG, zero_body, 0)

        def dma_start(b, s):
            pltpu.async_copy(x_hbm.at[pl.ds(b * R, R), :], xbs[s], sems[s])
            pltpu.async_copy(b_hbm.at[pl.ds(b * R, R)], bbs[s], sems[s])

        def dma_wait(b, s):
            pltpu.make_async_copy(
                x_hbm.at[pl.ds(b * R, R), :], xbs[s], sems[s]).wait()
            pltpu.make_async_copy(
                b_hbm.at[pl.ds(b * R, R)], bbs[s], sems[s]).wait()

        def _tree_group_sum(xb, row0, c):
            v = [xb[row0 + j, pl.ds(c * L, L)] for j in range(L)]
            while len(v) > 1:
                v = [v[i] + v[i + 1] for i in range(0, len(v), 2)]
            return v[0]

        def compute(s):
            xb, bb = xbs[s], bbs[s]
            bhead = bb[pl.ds(0, L)]
            btail = bb[pl.ds(R - L, L)]
            block_uniform = jnp.all(bhead == btail)

            @pl.when(block_uniform)
            def _uniform_block():
                def gb(g, carry):
                    return tuple(
                        carry[c] + _tree_group_sum(xb, g * L, c)
                        for c in range(F // L))

                tot = lax.fori_loop(
                    0, R // L, gb,
                    tuple(jnp.zeros((L,), jnp.float32) for _ in range(F // L)))
                for c in range(F // L):
                    plsc.addupdate_scatter(acc, [bhead, iota + c * L], tot[c])

            @pl.when(jnp.logical_not(block_uniform))
            def _mixed_block():
                def group_body(g, _):
                    row0 = g * L
                    bvec = bb[pl.ds(row0, L)]
                    b0 = bvec.at[jnp.zeros((L,), jnp.int32)].get(
                        mode="promise_in_bounds")
                    uniform = jnp.all(bvec == b0)

                    @pl.when(uniform)
                    def _fast():
                        for c in range(F // L):
                            v = _tree_group_sum(xb, row0, c)
                            plsc.addupdate_scatter(acc, [bvec, iota + c * L], v)

                    @pl.when(jnp.logical_not(uniform))
                    def _slow():
                        for j in range(L):
                            bj = bvec.at[jnp.full((L,), j, jnp.int32)].get(
                                mode="promise_in_bounds")
                            for c in range(F // L):
                                v = xb[row0 + j, pl.ds(c * L, L)]
                                plsc.addupdate_scatter(
                                    acc, [bj, iota + c * L], v)

                    return 0

                lax.fori_loop(0, R // L, group_body, 0)

        dma_start(wid, 0)
        for k in range(BLOCKS_PER_W):
            if (k + 1) < BLOCKS_PER_W:
                dma_start(wid + NW * (k + 1), (k + 1) % 2)
            dma_wait(wid + NW * k, k % 2)
            compute(k % 2)

        pltpu.sync_copy(acc, out_hbm.at[wid])

    return body(x, batch)


def _tc_partial(x, batch3):
    def body(b_ref, x_ref, out_ref):
        i = pl.program_id(0)

        @pl.when(i == 0)
        def _init():
            out_ref[...] = jnp.zeros_like(out_ref)

        seg = lax.broadcasted_iota(jnp.int32, (NUM_SEG, TC_CHUNK), 0)
        oh = (b_ref[0, 0, :][None, :] == seg).astype(jnp.float32)
        out_ref[...] += lax.dot_general(
            oh, x_ref[...], (((1,), (0,)), ((), ())),
            preferred_element_type=jnp.float32)

    return pl.pallas_call(
        body,
        grid=(TC_STEPS,),
        in_specs=[
            pl.BlockSpec((1, 1, TC_CHUNK),
                         lambda i: (N_SC // TC_CHUNK + i, 0, 0)),
            pl.BlockSpec((TC_CHUNK, F), lambda i: (N_SC // TC_CHUNK + i, 0)),
        ],
        out_specs=pl.BlockSpec((NUM_SEG, F), lambda i: (0, 0)),
        out_shape=jax.ShapeDtypeStruct((NUM_SEG, F), jnp.float32),
    )(batch3, x)


def _tc_combine(sc_partials, tc_partial, seed_nodes, W_rel, W_root, b_rel2):
    def body(p_ref, t_ref, seed_ref, wrel_ref, wroot_ref, brel_ref, out_ref):
        S = jnp.sum(p_ref[...], axis=0) + t_ref[...]
        A = lax.dot_general(S, wrel_ref[...], (((1,), (1,)), ((), ())),
                            preferred_element_type=jnp.float32)
        B = lax.dot_general(seed_ref[...], wroot_ref[...], (((1,), (1,)), ((), ())),
                            preferred_element_type=jnp.float32)
        o3 = A[:, None, :] + B[None, :, :] + brel_ref[...][None, :, :]
        out_ref[...] = o3.reshape(NUM_SEG * RATIO, F)

    return pl.pallas_call(
        body,
        out_shape=jax.ShapeDtypeStruct((NUM_SEG * RATIO, F), jnp.float32),
    )(sc_partials, tc_partial, seed_nodes, W_rel, W_root, b_rel2)


def kernel(x, batch, seed_nodes, W_rel, W_root, b_rel):
    batch = batch.astype(jnp.int32)
    sc_partials = _sc_segment_sum(x, batch)
    tc_part = _tc_partial(x, batch.reshape(N // TC_CHUNK, 1, TC_CHUNK))
    out = _tc_combine(sc_partials, tc_part, seed_nodes, W_rel, W_root,
                      b_rel.reshape(1, F))
    new_batchidx = jnp.asarray(
        np.repeat(np.arange(NUM_SEG, dtype=np.int32), RATIO))
    return out, new_batchidx


# speedup vs baseline: 1.2963x; 1.1144x over previous
import functools

import jax
import jax.numpy as jnp
import numpy as np
from jax import lax
from jax.experimental import pallas as pl
from jax.experimental.pallas import tpu as pltpu
from jax.experimental.pallas import tpu_sc as plsc

N = 100000
F = 128
NUM_SEG = 16
RATIO = 4

NC = 2
NS = 16
NW = NC * NS
L = 16

R = 400
N_SC = 48000
NB = N_SC // R
BLOCKS_PER_W = -(-NB // NW)

TC_CHUNK = 4000
N_TC = N - N_SC
TC_STEPS = N_TC // TC_CHUNK


def _sc_segment_sum(x, batch):
    mesh = plsc.VectorSubcoreMesh(core_axis_name="c", subcore_axis_name="s")

    @functools.partial(
        pl.kernel,
        mesh=mesh,
        out_type=jax.ShapeDtypeStruct((NW, NUM_SEG, F), jnp.float32),
        compiler_params=pltpu.CompilerParams(needs_layout_passes=False),
        scratch_types=[
            pltpu.VMEM((R, F), jnp.float32),
            pltpu.VMEM((R, F), jnp.float32),
            pltpu.VMEM((R,), jnp.int32),
            pltpu.VMEM((R,), jnp.int32),
            pltpu.VMEM((NUM_SEG, F), jnp.float32),
            pltpu.SemaphoreType.DMA,
            pltpu.SemaphoreType.DMA,
        ],
    )
    def body(x_hbm, b_hbm, out_hbm, xb0, xb1, bb0, bb1, acc, sem0, sem1):
        wid = lax.axis_index("s") * NC + lax.axis_index("c")
        iota = lax.broadcasted_iota(jnp.int32, (L,), 0)
        zeros = jnp.zeros((L,), jnp.float32)
        xbs, bbs, sems = (xb0, xb1), (bb0, bb1), (sem0, sem1)

        def zero_body(i, _):
            for c in range(F // L):
                acc[i, pl.ds(c * L, L)] = zeros
            return 0

        lax.fori_loop(0, NUM_SEG, zero_body, 0)

        def dma_start(b, s):
            pltpu.async_copy(x_hbm.at[pl.ds(b * R, R), :], xbs[s], sems[s])
            pltpu.async_copy(b_hbm.at[pl.ds(b * R, R)], bbs[s], sems[s])

        def dma_wait(b, s):
            pltpu.make_async_copy(
                x_hbm.at[pl.ds(b * R, R), :], xbs[s], sems[s]).wait()
            pltpu.make_async_copy(
                b_hbm.at[pl.ds(b * R, R)], bbs[s], sems[s]).wait()

        def _tree_group_sum(xb, row0, c):
            v = [xb[row0 + j, pl.ds(c * L, L)] for j in range(L)]
            while len(v) > 1:
                v = [v[i] + v[i + 1] for i in range(0, len(v), 2)]
            return v[0]

        def compute(s):
            xb, bb = xbs[s], bbs[s]
            bhead = bb[pl.ds(0, L)]
            btail = bb[pl.ds(R - L, L)]
            block_uniform = jnp.all(bhead == btail)

            @pl.when(block_uniform)
            def _uniform_block():
                def gb(g, carry):
                    return tuple(
                        carry[c] + _tree_group_sum(xb, g * L, c)
                        for c in range(F // L))

                tot = lax.fori_loop(
                    0, R // L, gb,
                    tuple(jnp.zeros((L,), jnp.float32) for _ in range(F // L)))
                for c in range(F // L):
                    plsc.addupdate_scatter(acc, [bhead, iota + c * L], tot[c])

            @pl.when(jnp.logical_not(block_uniform))
            def _mixed_block():
                def group_body(g, _):
                    row0 = g * L
                    bvec = bb[pl.ds(row0, L)]
                    b0 = bvec.at[jnp.zeros((L,), jnp.int32)].get(
                        mode="promise_in_bounds")
                    uniform = jnp.all(bvec == b0)

                    @pl.when(uniform)
                    def _fast():
                        for c in range(F // L):
                            v = _tree_group_sum(xb, row0, c)
                            plsc.addupdate_scatter(acc, [bvec, iota + c * L], v)

                    @pl.when(jnp.logical_not(uniform))
                    def _slow():
                        def row_body(j, _):
                            bj = bvec.at[jnp.full((L,), 0, jnp.int32) + j].get(
                                mode="promise_in_bounds")
                            for c in range(F // L):
                                v = xb[row0 + j, pl.ds(c * L, L)]
                                plsc.addupdate_scatter(
                                    acc, [bj, iota + c * L], v)
                            return 0

                        lax.fori_loop(0, L, row_body, 0)

                    return 0

                lax.fori_loop(0, R // L, group_body, 0)

        dma_start(wid, 0)
        for k in range(BLOCKS_PER_W):
            if (k + 1) < BLOCKS_PER_W:
                if NW * (k + 2) <= NB:
                    dma_start(wid + NW * (k + 1), (k + 1) % 2)
                else:
                    @pl.when(wid + NW * (k + 1) < NB)
                    def _pref():
                        dma_start(wid + NW * (k + 1), (k + 1) % 2)
            if NW * (k + 1) <= NB:
                dma_wait(wid + NW * k, k % 2)
                compute(k % 2)
            else:
                @pl.when(wid + NW * k < NB)
                def _tail():
                    dma_wait(wid + NW * k, k % 2)
                    compute(k % 2)

        pltpu.sync_copy(acc, out_hbm.at[wid])

    return body(x, batch)


def _tc_partial(x, batch3):
    def body(b_ref, x_ref, out_ref):
        i = pl.program_id(0)

        @pl.when(i == 0)
        def _init():
            out_ref[...] = jnp.zeros_like(out_ref)

        seg = lax.broadcasted_iota(jnp.int32, (NUM_SEG, TC_CHUNK), 0)
        oh = (b_ref[0, 0, :][None, :] == seg).astype(jnp.float32)
        out_ref[...] += lax.dot_general(
            oh, x_ref[...], (((1,), (0,)), ((), ())),
            preferred_element_type=jnp.float32)

    return pl.pallas_call(
        body,
        grid=(TC_STEPS,),
        in_specs=[
            pl.BlockSpec((1, 1, TC_CHUNK),
                         lambda i: (N_SC // TC_CHUNK + i, 0, 0)),
            pl.BlockSpec((TC_CHUNK, F), lambda i: (N_SC // TC_CHUNK + i, 0)),
        ],
        out_specs=pl.BlockSpec((NUM_SEG, F), lambda i: (0, 0)),
        out_shape=jax.ShapeDtypeStruct((NUM_SEG, F), jnp.float32),
    )(batch3, x)


def _tc_combine(sc_partials, tc_partial, seed_nodes, W_rel, W_root, b_rel2):
    def body(p_ref, t_ref, seed_ref, wrel_ref, wroot_ref, brel_ref, out_ref):
        S = jnp.sum(p_ref[...], axis=0) + t_ref[...]
        A = lax.dot_general(S, wrel_ref[...], (((1,), (1,)), ((), ())),
                            preferred_element_type=jnp.float32)
        B = lax.dot_general(seed_ref[...], wroot_ref[...], (((1,), (1,)), ((), ())),
                            preferred_element_type=jnp.float32)
        o3 = A[:, None, :] + B[None, :, :] + brel_ref[...][None, :, :]
        out_ref[...] = o3.reshape(NUM_SEG * RATIO, F)

    return pl.pallas_call(
        body,
        out_shape=jax.ShapeDtypeStruct((NUM_SEG * RATIO, F), jnp.float32),
    )(sc_partials, tc_partial, seed_nodes, W_rel, W_root, b_rel2)


def kernel(x, batch, seed_nodes, W_rel, W_root, b_rel):
    batch = batch.astype(jnp.int32)
    sc_partials = _sc_segment_sum(x, batch)
    tc_part = _tc_partial(x, batch.reshape(N // TC_CHUNK, 1, TC_CHUNK))
    out = _tc_combine(sc_partials, tc_part, seed_nodes, W_rel, W_root,
                      b_rel.reshape(1, F))
    new_batchidx = jnp.asarray(
        np.repeat(np.arange(NUM_SEG, dtype=np.int32), RATIO))
    return out, new_batchidx


# speedup vs baseline: 1.5031x; 1.1595x over previous
import functools

import jax
import jax.numpy as jnp
import numpy as np
from jax import lax
from jax.experimental import pallas as pl
from jax.experimental.pallas import tpu as pltpu
from jax.experimental.pallas import tpu_sc as plsc

N = 100000
F = 128
NUM_SEG = 16
RATIO = 4

NC = 2
NS = 16
NW = NC * NS
L = 16

R = 400
N_SC = 36000
NB = N_SC // R

TC_CHUNK = 4000
N_TC = N - N_SC
TC_STEPS = N_TC // TC_CHUNK


def _sc_segment_sum(x, batch):
    mesh = plsc.VectorSubcoreMesh(core_axis_name="c", subcore_axis_name="s")

    @functools.partial(
        pl.kernel,
        mesh=mesh,
        out_type=jax.ShapeDtypeStruct((NW, NUM_SEG, F), jnp.float32),
        compiler_params=pltpu.CompilerParams(needs_layout_passes=False),
        scratch_types=[
            pltpu.VMEM((2 * R, F), jnp.float32),
            pltpu.VMEM((2 * R,), jnp.int32),
            pltpu.VMEM((NUM_SEG, F), jnp.float32),
            pltpu.SemaphoreType.DMA,
            pltpu.SemaphoreType.DMA,
        ],
    )
    def body(x_hbm, b_hbm, out_hbm, xb, bb, acc, sem0, sem1):
        wid = lax.axis_index("s") * NC + lax.axis_index("c")
        iota = lax.broadcasted_iota(jnp.int32, (L,), 0)
        zeros = jnp.zeros((L,), jnp.float32)
        sems = (sem0, sem1)

        def zero_body(i, _):
            for c in range(F // L):
                acc[i, pl.ds(c * L, L)] = zeros
            return 0

        lax.fori_loop(0, NUM_SEG, zero_body, 0)

        def dma_start(b, s):
            pltpu.async_copy(x_hbm.at[pl.ds(b * R, R), :],
                             xb.at[pl.ds(s * R, R), :], sems[s])
            pltpu.async_copy(b_hbm.at[pl.ds(b * R, R)],
                             bb.at[pl.ds(s * R, R)], sems[s])

        def dma_wait(b, s):
            pltpu.make_async_copy(
                x_hbm.at[pl.ds(b * R, R), :],
                xb.at[pl.ds(s * R, R), :], sems[s]).wait()
            pltpu.make_async_copy(
                b_hbm.at[pl.ds(b * R, R)],
                bb.at[pl.ds(s * R, R)], sems[s]).wait()

        def _tree_group_sum(row0, c):
            v = [xb[row0 + j, pl.ds(c * L, L)] for j in range(L)]
            while len(v) > 1:
                v = [v[i] + v[i + 1] for i in range(0, len(v), 2)]
            return v[0]

        def compute(base):
            bhead = bb[pl.ds(base, L)]
            btail = bb[pl.ds(base + R - L, L)]
            block_uniform = jnp.all(bhead == btail)

            @pl.when(block_uniform)
            def _uniform_block():
                def gb(g, carry):
                    return tuple(
                        carry[c] + _tree_group_sum(base + g * L, c)
                        for c in range(F // L))

                tot = lax.fori_loop(
                    0, R // L, gb,
                    tuple(jnp.zeros((L,), jnp.float32) for _ in range(F // L)))
                for c in range(F // L):
                    plsc.addupdate_scatter(acc, [bhead, iota + c * L], tot[c])

            @pl.when(jnp.logical_not(block_uniform))
            def _mixed_block():
                def group_body(g, _):
                    row0 = base + g * L
                    bvec = bb[pl.ds(row0, L)]
                    b0 = bvec.at[jnp.zeros((L,), jnp.int32)].get(
                        mode="promise_in_bounds")
                    uniform = jnp.all(bvec == b0)

                    @pl.when(uniform)
                    def _fast():
                        for c in range(F // L):
                            v = _tree_group_sum(row0, c)
                            plsc.addupdate_scatter(acc, [bvec, iota + c * L], v)

                    @pl.when(jnp.logical_not(uniform))
                    def _slow():
                        def row_body(j, _):
                            bj = bvec.at[jnp.full((L,), 0, jnp.int32) + j].get(
                                mode="promise_in_bounds")
                            for c in range(F // L):
                                v = xb[row0 + j, pl.ds(c * L, L)]
                                plsc.addupdate_scatter(
                                    acc, [bj, iota + c * L], v)
                            return 0

                        lax.fori_loop(0, L, row_body, 0)

                    return 0

                lax.fori_loop(0, R // L, group_body, 0)

        nblk = (NB - 1 - wid) // NW + 1
        dma_start(wid, 0)

        def blk_body(k, _):
            b = wid + NW * k
            par = lax.rem(k, 2)
            has_next = b + NW < NB

            @pl.when(has_next & (par == 0))
            def _p0():
                dma_start(b + NW, 1)

            @pl.when(has_next & (par == 1))
            def _p1():
                dma_start(b + NW, 0)

            @pl.when(par == 0)
            def _w0():
                dma_wait(b, 0)

            @pl.when(par == 1)
            def _w1():
                dma_wait(b, 1)

            compute(par * R)
            return 0

        lax.fori_loop(0, nblk, blk_body, 0)

        pltpu.sync_copy(acc, out_hbm.at[wid])

    return body(x, batch)


def _tc_partial(x, batch3):
    def body(b_ref, x_ref, out_ref):
        i = pl.program_id(0)

        @pl.when(i == 0)
        def _init():
            out_ref[...] = jnp.zeros_like(out_ref)

        seg = lax.broadcasted_iota(jnp.int32, (NUM_SEG, TC_CHUNK), 0)
        oh = (b_ref[0, 0, :][None, :] == seg).astype(jnp.float32)
        out_ref[...] += lax.dot_general(
            oh, x_ref[...], (((1,), (0,)), ((), ())),
            preferred_element_type=jnp.float32)

    return pl.pallas_call(
        body,
        grid=(TC_STEPS,),
        in_specs=[
            pl.BlockSpec((1, 1, TC_CHUNK),
                         lambda i: (N_SC // TC_CHUNK + i, 0, 0)),
            pl.BlockSpec((TC_CHUNK, F), lambda i: (N_SC // TC_CHUNK + i, 0)),
        ],
        out_specs=pl.BlockSpec((NUM_SEG, F), lambda i: (0, 0)),
        out_shape=jax.ShapeDtypeStruct((NUM_SEG, F), jnp.float32),
    )(batch3, x)


def _tc_combine(sc_partials, tc_partial, seed_nodes, W_rel, W_root, b_rel2):
    def body(p_ref, t_ref, seed_ref, wrel_ref, wroot_ref, brel_ref, out_ref):
        S = jnp.sum(p_ref[...], axis=0) + t_ref[...]
        A = lax.dot_general(S, wrel_ref[...], (((1,), (1,)), ((), ())),
                            preferred_element_type=jnp.float32)
        B = lax.dot_general(seed_ref[...], wroot_ref[...], (((1,), (1,)), ((), ())),
                            preferred_element_type=jnp.float32)
        o3 = A[:, None, :] + B[None, :, :] + brel_ref[...][None, :, :]
        out_ref[...] = o3.reshape(NUM_SEG * RATIO, F)

    return pl.pallas_call(
        body,
        out_shape=jax.ShapeDtypeStruct((NUM_SEG * RATIO, F), jnp.float32),
    )(sc_partials, tc_partial, seed_nodes, W_rel, W_root, b_rel2)


def kernel(x, batch, seed_nodes, W_rel, W_root, b_rel):
    batch = batch.astype(jnp.int32)
    sc_partials = _sc_segment_sum(x, batch)
    tc_part = _tc_partial(x, batch.reshape(N // TC_CHUNK, 1, TC_CHUNK))
    out = _tc_combine(sc_partials, tc_part, seed_nodes, W_rel, W_root,
                      b_rel.reshape(1, F))
    new_batchidx = jnp.asarray(
        np.repeat(np.arange(NUM_SEG, dtype=np.int32), RATIO))
    return out, new_batchidx
